# Initial kernel scaffold; baseline (speedup 1.0000x reference)
import jax
import jax.numpy as jnp
from jax.experimental import pallas as pl


def kernel(x, w1, w2, router_w, bias):
    raise NotImplementedError("write your pallas kernel here")


# speedup vs baseline: 1.4676x; 1.4676x over previous
import functools

import jax
import jax.numpy as jnp
from jax.experimental import pallas as pl
from jax.experimental.pallas import tpu as pltpu


def _router_body(x_ref, rw_ref, coef_ref):
    x = x_ref[...]
    logits = jnp.dot(x, rw_ref[...].T, preferred_element_type=jnp.float32)
    p = jax.nn.softmax(logits, axis=-1)
    eidx = jax.lax.broadcasted_iota(jnp.int32, p.shape, 1)
    m1 = jnp.max(p, axis=-1, keepdims=True)
    i1 = jnp.argmax(p, axis=-1)[:, None]
    p_masked = jnp.where(eidx == i1, -jnp.inf, p)
    m2 = jnp.max(p_masked, axis=-1, keepdims=True)
    i2 = jnp.argmax(p_masked, axis=-1)[:, None]
    denom = m1 + m2
    coef = jnp.where(eidx == i1, m1 / denom, jnp.where(eidx == i2, m2 / denom, 0.0))
    coef_ref[...] = coef.T[:, None, :]


def _moe_body(x_ref, w1_ref, w2_ref, coef_ref, bias_ref, out_ref, *, H, E):
    e = pl.program_id(1)

    @pl.when(e == 0)
    def _init():
        out_ref[...] = jnp.zeros_like(out_ref)

    h = jnp.dot(x_ref[...], w1_ref[0].T, preferred_element_type=jnp.float32)
    g = h[:, :H]
    u = h[:, H:]
    act = g * jax.nn.sigmoid(g) * u
    y = jnp.dot(act, w2_ref[0].T, preferred_element_type=jnp.float32)
    coef = coef_ref[0, 0, :][:, None]
    out_ref[...] += y * coef

    @pl.when(e == E - 1)
    def _bias():
        out_ref[...] += bias_ref[...][None, :]


@jax.jit
def kernel(x, w1, w2, router_w, bias):
    T, D = x.shape
    E, H2, _ = w1.shape
    H = H2 // 2

    coef = pl.pallas_call(
        _router_body,
        out_shape=jax.ShapeDtypeStruct((E, 1, T), jnp.float32),
        in_specs=[
            pl.BlockSpec((T, D), lambda: (0, 0)),
            pl.BlockSpec((E, D), lambda: (0, 0)),
        ],
        out_specs=pl.BlockSpec((E, 1, T), lambda: (0, 0, 0)),
    )(x, router_w)

    BT = 256
    NT = T // BT
    out = pl.pallas_call(
        functools.partial(_moe_body, H=H, E=E),
        grid=(NT, E),
        out_shape=jax.ShapeDtypeStruct((T, D), jnp.float32),
        in_specs=[
            pl.BlockSpec((BT, D), lambda t, e: (t, 0)),
            pl.BlockSpec((1, H2, D), lambda t, e: (e, 0, 0)),
            pl.BlockSpec((1, D, H), lambda t, e: (e, 0, 0)),
            pl.BlockSpec((1, 1, BT), lambda t, e: (e, 0, t)),
            pl.BlockSpec((D,), lambda t, e: (0,)),
        ],
        out_specs=pl.BlockSpec((BT, D), lambda t, e: (t, 0)),
        compiler_params=pltpu.CompilerParams(
            dimension_semantics=("arbitrary", "arbitrary"),
        ),
    )(x, w1, w2, coef, bias)
    return out


# speedup vs baseline: 2.5127x; 1.7122x over previous
import functools

import jax
import jax.numpy as jnp
from jax import lax
from jax.experimental import pallas as pl
from jax.experimental.pallas import tpu as pltpu
from jax.experimental.pallas import tpu_sc as plsc

_B = 256
_NW = 32


def _cumsum_lanes(a, n):
    sh = 1
    while sh < n:
        z = jnp.zeros(a.shape[:1] + (sh,), a.dtype)
        a = a + jnp.concatenate([z, a[:, :-sh]], axis=1)
        sh *= 2
    return a


def _cumsum_subl(a, n):
    sh = 1
    while sh < n:
        z = jnp.zeros((sh,) + a.shape[1:], a.dtype)
        a = a + jnp.concatenate([z, a[:-sh, :]], axis=0)
        sh *= 2
    return a


def _router_body(x_ref, rw_ref, slots_ref, wnT_ref, be_ref, *, T, E, NB):
    x = x_ref[...]
    logits = jnp.dot(x, rw_ref[...].T, preferred_element_type=jnp.float32)
    pT = jax.nn.softmax(logits, axis=-1).T
    eiota = lax.broadcasted_iota(jnp.int32, (E, T), 0)
    m1 = jnp.max(pT, axis=0, keepdims=True)
    i1 = jnp.min(jnp.where(pT == m1, eiota, E), axis=0, keepdims=True)
    pm = jnp.where(eiota == i1, -jnp.inf, pT)
    m2 = jnp.max(pm, axis=0, keepdims=True)
    i2 = jnp.min(jnp.where(pm == m2, eiota, E), axis=0, keepdims=True)
    denom = m1 + m2
    wnT_ref[...] = jnp.concatenate([m1 / denom, m2 / denom], axis=0)

    sel = ((eiota == i1) | (eiota == i2)).astype(jnp.int32)
    csum = _cumsum_lanes(sel, T)
    cnt = csum[:, T - 1 : T]
    nblk = (cnt + (_B - 1)) // _B
    blk_incl = _cumsum_subl(nblk, E)
    base = _B * (blk_incl - nblk)
    v = base + csum - sel
    slot1 = jnp.sum(jnp.where(eiota == i1, v, 0), axis=0, keepdims=True)
    slot2 = jnp.sum(jnp.where(eiota == i2, v, 0), axis=0, keepdims=True)
    slots_ref[...] = jnp.concatenate([slot1, slot2], axis=0)

    biota = lax.broadcasted_iota(jnp.int32, (E, 128), 1)
    be = jnp.sum((biota >= blk_incl).astype(jnp.int32), axis=0, keepdims=True)
    be = jnp.minimum(be, E - 1)
    liota = lax.broadcasted_iota(jnp.int32, (1, 128), 1)
    be_ref[...] = jnp.where(liota == 31, blk_incl[E - 1 : E, :], be)


def _gmm_body(be_ref, xs_ref, w1_ref, w2_ref, ys_ref, *, H):
    b = pl.program_id(0)

    @pl.when(b < be_ref[31])
    def _live():
        h = jnp.dot(xs_ref[...], w1_ref[0].T, preferred_element_type=jnp.float32)
        g = h[:, :H]
        u = h[:, H:]
        act = g * jax.nn.sigmoid(g) * u
        ys_ref[...] = jnp.dot(act, w2_ref[0].T, preferred_element_type=jnp.float32)


def _combine_body(yk_ref, wnT_ref, bias_ref, out_ref):
    a = yk_ref[0]
    b = yk_ref[1]
    w1c = wnT_ref[0, :][:, None]
    w2c = wnT_ref[1, :][:, None]
    out_ref[...] = w1c * a + w2c * b + bias_ref[...][None, :]


@jax.jit
def kernel(x, w1, w2, router_w, bias):
    T, D = x.shape
    E, H2, _ = w1.shape
    H = H2 // 2
    K = 2
    NB = (T * K + E * (_B - 1)) // _B
    S = NB * _B
    TPW = T // _NW

    slots, wnT, be = pl.pallas_call(
        functools.partial(_router_body, T=T, E=E, NB=NB),
        out_shape=(
            jax.ShapeDtypeStruct((K, T), jnp.int32),
            jax.ShapeDtypeStruct((K, T), jnp.float32),
            jax.ShapeDtypeStruct((1, 128), jnp.int32),
        ),
        in_specs=[
            pl.BlockSpec((T, D), lambda: (0, 0)),
            pl.BlockSpec((E, D), lambda: (0, 0)),
        ],
        out_specs=(
            pl.BlockSpec((K, T), lambda: (0, 0)),
            pl.BlockSpec((K, T), lambda: (0, 0)),
            pl.BlockSpec((1, 128), lambda: (0, 0)),
        ),
    )(x, router_w)

    mesh = plsc.VectorSubcoreMesh(core_axis_name="c", subcore_axis_name="s")

    @functools.partial(
        pl.kernel,
        mesh=mesh,
        out_type=jax.ShapeDtypeStruct((S, D), jnp.float32),
        scratch_types=[
            pltpu.VMEM((TPW,), jnp.int32),
            pltpu.VMEM((TPW,), jnp.int32),
            pltpu.VMEM((TPW, D), jnp.float32),
            pltpu.SemaphoreType.DMA,
            pltpu.SemaphoreType.DMA,
        ],
    )
    def _dispatch(x_hbm, slots_hbm, xs_hbm, idx1_v, idx2_v, xbuf, sem1, sem2):
        wid = lax.axis_index("s") * 2 + lax.axis_index("c")
        base = wid * TPW
        pltpu.sync_copy(slots_hbm.at[0, pl.ds(base, TPW)], idx1_v)
        pltpu.sync_copy(slots_hbm.at[1, pl.ds(base, TPW)], idx2_v)
        pltpu.sync_copy(x_hbm.at[pl.ds(base, TPW)], xbuf)
        c1 = pltpu.async_copy(xbuf, xs_hbm.at[idx1_v], sem1)
        c2 = pltpu.async_copy(xbuf, xs_hbm.at[idx2_v], sem2)
        c1.wait()
        c2.wait()

    xs = _dispatch(x, slots)

    grid_spec = pltpu.PrefetchScalarGridSpec(
        num_scalar_prefetch=1,
        grid=(NB,),
        in_specs=[
            pl.BlockSpec((_B, D), lambda b, be_s: (b, 0)),
            pl.BlockSpec((1, H2, D), lambda b, be_s: (be_s[b], 0, 0)),
            pl.BlockSpec((1, D, H), lambda b, be_s: (be_s[b], 0, 0)),
        ],
        out_specs=pl.BlockSpec((_B, D), lambda b, be_s: (b, 0)),
    )
    ys = pl.pallas_call(
        functools.partial(_gmm_body, H=H),
        grid_spec=grid_spec,
        out_shape=jax.ShapeDtypeStruct((S, D), jnp.float32),
        compiler_params=pltpu.CompilerParams(
            dimension_semantics=("arbitrary",),
        ),
    )(be.reshape(128), xs, w1, w2)

    @functools.partial(
        pl.kernel,
        mesh=mesh,
        out_type=jax.ShapeDtypeStruct((K, T, D), jnp.float32),
        scratch_types=[
            pltpu.VMEM((TPW,), jnp.int32),
            pltpu.VMEM((TPW,), jnp.int32),
            pltpu.VMEM((TPW, D), jnp.float32),
            pltpu.VMEM((TPW, D), jnp.float32),
            pltpu.SemaphoreType.DMA,
            pltpu.SemaphoreType.DMA,
        ],
    )
    def _unperm(ys_hbm, slots_hbm, yk_hbm, idx1_v, idx2_v, buf1, buf2, sem1, sem2):
        wid = lax.axis_index("s") * 2 + lax.axis_index("c")
        base = wid * TPW
        pltpu.sync_copy(slots_hbm.at[0, pl.ds(base, TPW)], idx1_v)
        pltpu.sync_copy(slots_hbm.at[1, pl.ds(base, TPW)], idx2_v)
        c1 = pltpu.async_copy(ys_hbm.at[idx1_v], buf1, sem1)
        c2 = pltpu.async_copy(ys_hbm.at[idx2_v], buf2, sem2)
        c1.wait()
        pltpu.sync_copy(buf1, yk_hbm.at[0, pl.ds(base, TPW)])
        c2.wait()
        pltpu.sync_copy(buf2, yk_hbm.at[1, pl.ds(base, TPW)])

    yk = _unperm(ys, slots)

    BT = 256
    out = pl.pallas_call(
        _combine_body,
        grid=(T // BT,),
        out_shape=jax.ShapeDtypeStruct((T, D), jnp.float32),
        in_specs=[
            pl.BlockSpec((K, BT, D), lambda t: (0, t, 0)),
            pl.BlockSpec((K, BT), lambda t: (0, t)),
            pl.BlockSpec((D,), lambda t: (0,)),
        ],
        out_specs=pl.BlockSpec((BT, D), lambda t: (t, 0)),
    )(yk, wnT, bias)
    return out
